# Initial kernel scaffold; baseline (speedup 1.0000x reference)
import jax
import jax.numpy as jnp
from jax.experimental import pallas as pl


def kernel(x, edge_index, W1, b1, W2, b2):
    raise NotImplementedError("write your pallas kernel here")


# speedup vs baseline: 14.2116x; 14.2116x over previous
import functools

import jax
import jax.numpy as jnp
from jax import lax
from jax.experimental import pallas as pl
from jax.experimental.pallas import tpu as pltpu
from jax.experimental.pallas import tpu_sc as plsc

_N = 10000
_E = 320000
_D = 128
_NC = 2
_NS = 16
_L = 16

_NP = 10240
_CHK = _NP // _NS

_EPT = _E // _NS

_EPC = _E // _NC
_EPT2 = _EPC // _NS
_K = 80
_NB = _EPT2 // _K
_RPT = _NP // _NS
_ZR = 128

_sc_mesh = plsc.VectorSubcoreMesh(core_axis_name="c", subcore_axis_name="s")


@functools.partial(
    pl.kernel,
    out_type=jax.ShapeDtypeStruct((_NP,), jnp.float32),
    mesh=_sc_mesh,
    scratch_types=[
        pltpu.VMEM((_EPT,), jnp.int32),
        pltpu.VMEM((_NP,), jnp.float32),
        pltpu.VMEM((_NS, _CHK), jnp.float32),
        pltpu.VMEM((_CHK,), jnp.float32),
        pltpu.VMEM_SHARED((_NS, _NP), jnp.float32),
    ],
    compiler_params=pltpu.CompilerParams(needs_layout_passes=False),
)
def _deg_kernel(dst_hbm, dinv_hbm, idx_v, hist_v, buf, dinv_v, deg_sh):
    c = lax.axis_index("c")
    s = lax.axis_index("s")

    zeros = jnp.zeros((_L,), jnp.float32)

    def zero_body(i, carry):
        hist_v[pl.ds(i * _L, _L)] = zeros
        return carry

    lax.fori_loop(0, _NP // _L, zero_body, 0)

    pltpu.sync_copy(dst_hbm.at[pl.ds(s * _EPT, _EPT)], idx_v)
    ones = jnp.ones((_L,), jnp.float32)

    def hist_body(i, carry):
        idx = idx_v[pl.ds(i * _L, _L)]
        plsc.addupdate_scatter(hist_v, [idx], ones)
        return carry

    lax.fori_loop(0, _EPT // _L, hist_body, 0)

    pltpu.sync_copy(hist_v, deg_sh.at[s])
    plsc.subcore_barrier()
    for t in range(_NS):
        pltpu.sync_copy(deg_sh.at[t, pl.ds(s * _CHK, _CHK)], buf.at[t])

    half = jnp.full((_L,), 0.5, jnp.float32)
    three_half = jnp.full((_L,), 1.5, jnp.float32)
    magic = jnp.full((_L,), 0x5F3759DF, jnp.int32)

    def red_body(j, carry):
        v = buf[0, pl.ds(j * _L, _L)]
        for t in range(1, _NS):
            v = v + buf[t, pl.ds(j * _L, _L)]
        d = v + 1.0
        bits = plsc.bitcast(d, jnp.int32)
        y = plsc.bitcast(magic - (bits >> 1), jnp.float32)
        for _ in range(3):
            y = y * (three_half - half * d * y * y)
        dinv_v[pl.ds(j * _L, _L)] = y
        return carry

    lax.fori_loop(0, _CHK // _L, red_body, 0)

    @pl.when(c == 0)
    def _():
        pltpu.sync_copy(dinv_v, dinv_hbm.at[pl.ds(s * _CHK, _CHK)])


@functools.partial(
    pl.kernel,
    out_type=(
        jax.ShapeDtypeStruct((_NP, _D), jnp.float32),
        jax.ShapeDtypeStruct((_NP, _D), jnp.float32),
    ),
    mesh=_sc_mesh,
    scratch_types=[
        pltpu.VMEM((_K,), jnp.int32),
        pltpu.VMEM((_K,), jnp.int32),
        pltpu.VMEM((_K, _D), jnp.float32),
        pltpu.VMEM((_ZR, _D), jnp.float32),
        pltpu.VMEM_SHARED((_NP, _D), jnp.float32),
        pltpu.SemaphoreType.DMA,
    ],
    compiler_params=pltpu.CompilerParams(needs_layout_passes=False),
)
def _prop_kernel(hp_hbm, src_hbm, dst_hbm, out0_hbm, out1_hbm, idx_s, idx_d, rows, zbuf, acc_sh, sem):
    c = lax.axis_index("c")
    s = lax.axis_index("s")

    zeros = jnp.zeros((_L,), jnp.float32)

    def zfill(r, carry):
        for j in range(_D // _L):
            zbuf[r, pl.ds(j * _L, _L)] = zeros
        return carry

    lax.fori_loop(0, _ZR, zfill, 0)
    row0 = s * _RPT
    for t in range(_RPT // _ZR):
        pltpu.sync_copy(zbuf, acc_sh.at[pl.ds(row0 + t * _ZR, _ZR)])
    plsc.subcore_barrier()

    ebase = c * _EPC + s * _EPT2

    def body(i, carry):
        b = ebase + i * _K
        pltpu.sync_copy(src_hbm.at[pl.ds(b, _K)], idx_s)
        pltpu.async_copy(hp_hbm.at[idx_s], rows, sem).wait()
        pltpu.sync_copy(dst_hbm.at[pl.ds(b, _K)], idx_d)
        pltpu.sync_copy(rows, acc_sh.at[idx_d], add=True)
        return carry

    lax.fori_loop(0, _NB, body, 0)
    plsc.subcore_barrier()

    @pl.when(c == 0)
    def _():
        for t in range(_RPT // _ZR):
            pltpu.sync_copy(
                acc_sh.at[pl.ds(row0 + t * _ZR, _ZR)],
                out0_hbm.at[pl.ds(row0 + t * _ZR, _ZR)],
            )

    @pl.when(c == 1)
    def _():
        for t in range(_RPT // _ZR):
            pltpu.sync_copy(
                acc_sh.at[pl.ds(row0 + t * _ZR, _ZR)],
                out1_hbm.at[pl.ds(row0 + t * _ZR, _ZR)],
            )


_BN = 1000
_G = _N // _BN


def _tc1_body(dinv_ref, x_ref, w1_ref, hp_ref):
    h = jnp.dot(x_ref[...], w1_ref[...], preferred_element_type=jnp.float32)
    hp_ref[...] = dinv_ref[...] * h


def _tc2_body(a0_ref, a1_ref, hp_ref, dinv_ref, b1_ref, w2_ref, hp2_ref):
    dinv = dinv_ref[...]
    z = dinv * (a0_ref[...] + a1_ref[...] + hp_ref[...]) + b1_ref[...]
    z = jnp.maximum(z, 0.0)
    hp2_ref[...] = dinv * jnp.dot(z, w2_ref[...], preferred_element_type=jnp.float32)


def _tc3_body(a0_ref, a1_ref, hp2_ref, dinv_ref, b2_ref, out_ref):
    out_ref[...] = (
        dinv_ref[...] * (a0_ref[...] + a1_ref[...] + hp2_ref[...]) + b2_ref[...]
    )


_tc1 = pl.pallas_call(
    _tc1_body,
    grid=(_G,),
    in_specs=[
        pl.BlockSpec((_BN, 1), lambda i: (i, 0)),
        pl.BlockSpec((_BN, _D), lambda i: (i, 0)),
        pl.BlockSpec((_D, _D), lambda i: (0, 0)),
    ],
    out_specs=pl.BlockSpec((_BN, _D), lambda i: (i, 0)),
    out_shape=jax.ShapeDtypeStruct((_N, _D), jnp.float32),
)

_tc2 = pl.pallas_call(
    _tc2_body,
    grid=(_G,),
    in_specs=[
        pl.BlockSpec((_BN, _D), lambda i: (i, 0)),
        pl.BlockSpec((_BN, _D), lambda i: (i, 0)),
        pl.BlockSpec((_BN, _D), lambda i: (i, 0)),
        pl.BlockSpec((_BN, 1), lambda i: (i, 0)),
        pl.BlockSpec((1, _D), lambda i: (0, 0)),
        pl.BlockSpec((_D, _D), lambda i: (0, 0)),
    ],
    out_specs=pl.BlockSpec((_BN, _D), lambda i: (i, 0)),
    out_shape=jax.ShapeDtypeStruct((_N, _D), jnp.float32),
)

_tc3 = pl.pallas_call(
    _tc3_body,
    grid=(_G,),
    in_specs=[
        pl.BlockSpec((_BN, _D), lambda i: (i, 0)),
        pl.BlockSpec((_BN, _D), lambda i: (i, 0)),
        pl.BlockSpec((_BN, _D), lambda i: (i, 0)),
        pl.BlockSpec((_BN, 1), lambda i: (i, 0)),
        pl.BlockSpec((1, _D), lambda i: (0, 0)),
    ],
    out_specs=pl.BlockSpec((_BN, _D), lambda i: (i, 0)),
    out_shape=jax.ShapeDtypeStruct((_N, _D), jnp.float32),
)


def kernel(x, edge_index, W1, b1, W2, b2):
    src = edge_index[0].astype(jnp.int32)
    dst = edge_index[1].astype(jnp.int32)
    b1r = b1.reshape(1, _D)
    b2r = b2.reshape(1, _D)

    dinv = _deg_kernel(dst)[: _N].reshape(_N, 1)
    hp1 = _tc1(dinv, x, W1)
    a0, a1 = _prop_kernel(hp1, src, dst)
    hp2 = _tc2(a0, a1, hp1, dinv, b1r, W2)
    c0, c1 = _prop_kernel(hp2, src, dst)
    out = _tc3(c0, c1, hp2, dinv, b2r)
    return out
